# Initial kernel scaffold; baseline (speedup 1.0000x reference)
import jax
import jax.numpy as jnp
from jax.experimental import pallas as pl


def kernel(x, level_hv, W):
    raise NotImplementedError("write your pallas kernel here")


# speedup vs baseline: 16.7331x; 16.7331x over previous
import functools

import jax
import jax.numpy as jnp
from jax import lax
from jax.experimental import pallas as pl
from jax.experimental.pallas import tpu as pltpu
from jax.experimental.pallas import tpu_sc as plsc

DIM = 4096
NLEV = 21
NCHUNK = 8
LANES = 16
NJ = 598
NJ_GROUPS = 74
NJ_REM = 6
XPAD = 608
NCLS = 5
THRESH = 299


def _csa(a, b, cin):
    u = a ^ b
    return u ^ cin, (a & b) | (u & cin)


def _sc_body(xin, ptab, wp, out, xin_v, idx_v, ptab_v, wp_v, out_v):
    wid = lax.axis_index("s") * 2 + lax.axis_index("c")
    pltpu.sync_copy(xin.at[wid], xin_v)
    pltpu.sync_copy(ptab, ptab_v)
    pltpu.sync_copy(wp, wp_v)

    for t in range(XPAD // LANES):
        v = xin_v[pl.ds(t * LANES, LANES)]
        u = (v / 20.0) * 20.0
        h = u + 0.5
        r = h.astype(jnp.int32)
        is_half = r.astype(jnp.float32) == h
        r = r - jnp.where(is_half, r & 1, 0)
        r = jnp.minimum(jnp.maximum(r, 0), NLEV - 1)
        idx_v[pl.ds(t * LANES, LANES)] = r

    def product(va, k, c):
        ia = va[k]
        ib = va[k + 1]
        ic = va[k + 2]
        w2 = ptab_v[pl.ds((ia * NCHUNK + c) * LANES, LANES)]
        w1 = ptab_v[pl.ds(((NLEV + ib) * NCHUNK + c) * LANES, LANES)]
        w0 = ptab_v[pl.ds(((2 * NLEV + ic) * NCHUNK + c) * LANES, LANES)]
        return (w2 ^ w1) ^ w0

    acc = [jnp.zeros((LANES,), jnp.float32) for _ in range(NCLS)]
    zero = jnp.zeros((LANES,), jnp.int32)

    for c in range(NCHUNK):
        def group(g, st, c=c):
            ones, twos, fours, p3, p4, p5, p6, p7, p8, p9 = st
            va = idx_v[pl.ds(g * 8, LANES)]
            x = [product(va, k, c) for k in range(8)]
            s0, c0 = _csa(x[0], x[1], x[2])
            s1, c1 = _csa(x[3], x[4], x[5])
            s2, c2 = _csa(x[6], x[7], s0)
            ones, c3 = _csa(s1, s2, ones)
            t0, d0 = _csa(c0, c1, c2)
            twos, d1 = _csa(c3, t0, twos)
            fours, e0 = _csa(d0, d1, fours)
            carry = e0
            ps = [p3, p4, p5, p6, p7, p8, p9]
            for i in range(7):
                nxt = ps[i] ^ carry
                carry = ps[i] & carry
                ps[i] = nxt
            return (ones, twos, fours, *ps)

        planes = list(lax.fori_loop(0, NJ_GROUPS, group, (zero,) * 10))

        vrem = idx_v[pl.ds(NJ_GROUPS * 8, LANES)]
        for k in range(NJ_REM):
            carry = product(vrem, k, c)
            for i in range(10):
                nxt = planes[i] ^ carry
                carry = planes[i] & carry
                planes[i] = nxt

        kadd = 1024 - THRESH
        carry = zero
        for p in range(10):
            if (kadd >> p) & 1:
                carry = planes[p] | carry
            else:
                carry = planes[p] & carry
        sbits = carry

        def clsbody(bit, accs, c=c, sbits=sbits):
            m = jnp.right_shift(sbits, bit) & 1
            e = 1.0 - 2.0 * m.astype(jnp.float32)
            return tuple(
                a + e * wp_v[pl.ds(((k * NCHUNK + c) * 32 + bit) * LANES, LANES)]
                for k, a in enumerate(accs))

        acc = list(lax.fori_loop(0, 32, clsbody, tuple(acc)))

    io = lax.broadcasted_iota(jnp.int32, (LANES,), 0)
    ov = jnp.zeros((LANES,), jnp.float32)
    for k in range(NCLS):
        total = acc[k][0]
        for l in range(1, LANES):
            total = total + acc[k][l]
        ov = jnp.where(io == k, total, ov)
    out_v[...] = ov
    pltpu.sync_copy(out_v, out.at[wid])


def _pack_signs(sgn):
    b = sgn.reshape(NLEV, NCHUNK, LANES, 32)
    w = jnp.sum(b << jnp.arange(32, dtype=jnp.uint32), axis=-1, dtype=jnp.uint32)
    return lax.bitcast_convert_type(w, jnp.int32)


def kernel(x, level_hv, W):
    batch = x.shape[0]
    flat = x.reshape(batch, -1)
    xin = jnp.pad(flat, ((0, 0), (0, XPAD - flat.shape[1])))

    sgn = (level_hv < 0).astype(jnp.uint32)
    ptab = jnp.stack([
        _pack_signs(jnp.roll(sgn, 2, axis=-1)),
        _pack_signs(jnp.roll(sgn, 1, axis=-1)),
        _pack_signs(sgn),
    ])
    wp = W.reshape(NCLS, NCHUNK, LANES, 32).transpose(0, 1, 3, 2)

    mesh = plsc.VectorSubcoreMesh(core_axis_name="c", subcore_axis_name="s")
    out = pl.kernel(
        _sc_body,
        mesh=mesh,
        out_type=jax.ShapeDtypeStruct((batch, LANES), jnp.float32),
        scratch_types=[
            pltpu.VMEM((XPAD,), jnp.float32),
            pltpu.VMEM((XPAD,), jnp.int32),
            pltpu.VMEM((3 * NLEV * NCHUNK * LANES,), jnp.int32),
            pltpu.VMEM((NCLS * NCHUNK * 32 * LANES,), jnp.float32),
            pltpu.VMEM((LANES,), jnp.float32),
        ],
    )(xin, ptab.reshape(-1), wp.reshape(-1))
    return out[:, :NCLS]


# speedup vs baseline: 16.7724x; 1.0023x over previous
import functools

import jax
import jax.numpy as jnp
from jax import lax
from jax.experimental import pallas as pl
from jax.experimental.pallas import tpu as pltpu
from jax.experimental.pallas import tpu_sc as plsc

DIM = 4096
NLEV = 21
NCHUNK = 8
LANES = 16
NJ = 598
NJ_GROUPS = 74
NJ_REM = 6
XPAD = 608
NCLS = 5
THRESH = 299


def _csa(a, b, cin):
    u = a ^ b
    return u ^ cin, (a & b) | (u & cin)


def _sc_body(xin, ptab, wp, out, xin_v, ptab_v, wp_v, out_v, offs_s):
    wid = lax.axis_index("s") * 2 + lax.axis_index("c")
    pltpu.sync_copy(xin.at[wid], xin_v)
    pltpu.sync_copy(ptab, ptab_v)
    pltpu.sync_copy(wp, wp_v)

    for t in range(XPAD // LANES):
        v = xin_v[pl.ds(t * LANES, LANES)]
        u = (v / 20.0) * 20.0
        h = u + 0.5
        r = h.astype(jnp.int32)
        is_half = r.astype(jnp.float32) == h
        r = r - jnp.where(is_half, r & 1, 0)
        r = jnp.minimum(jnp.maximum(r, 0), NLEV - 1)
        roff = r * (NCHUNK * LANES)
        for l in range(LANES):
            offs_s[t * LANES + l] = roff[l]

    def product(ovals, k, c):
        w2 = ptab_v[pl.ds(ovals[k] + c * LANES, LANES)]
        w1 = ptab_v[pl.ds(ovals[k + 1] + (NLEV * NCHUNK + c) * LANES, LANES)]
        w0 = ptab_v[pl.ds(ovals[k + 2] + (2 * NLEV * NCHUNK + c) * LANES, LANES)]
        return (w2 ^ w1) ^ w0

    acc = [jnp.zeros((LANES,), jnp.float32) for _ in range(NCLS)]
    zero = jnp.zeros((LANES,), jnp.int32)

    for c in range(NCHUNK):
        def group(g, st, c=c):
            ones, twos, fours, p3, p4, p5, p6, p7, p8, p9 = st
            j0 = g * 8
            ovals = [offs_s[j0 + i] for i in range(10)]
            x = [product(ovals, k, c) for k in range(8)]
            s0, c0 = _csa(x[0], x[1], x[2])
            s1, c1 = _csa(x[3], x[4], x[5])
            s2, c2 = _csa(x[6], x[7], s0)
            ones, c3 = _csa(s1, s2, ones)
            t0, d0 = _csa(c0, c1, c2)
            twos, d1 = _csa(c3, t0, twos)
            fours, e0 = _csa(d0, d1, fours)
            carry = e0
            ps = [p3, p4, p5, p6, p7, p8, p9]
            for i in range(7):
                nxt = ps[i] ^ carry
                carry = ps[i] & carry
                ps[i] = nxt
            return (ones, twos, fours, *ps)

        planes = list(lax.fori_loop(0, NJ_GROUPS, group, (zero,) * 10))

        orem = [offs_s[NJ_GROUPS * 8 + i] for i in range(NJ_REM + 2)]
        for k in range(NJ_REM):
            carry = product(orem, k, c)
            for i in range(10):
                nxt = planes[i] ^ carry
                carry = planes[i] & carry
                planes[i] = nxt

        kadd = 1024 - THRESH
        carry = zero
        for p in range(10):
            if (kadd >> p) & 1:
                carry = planes[p] | carry
            else:
                carry = planes[p] & carry
        sbits = carry

        def clsbody(bit, accs, c=c, sbits=sbits):
            m = jnp.right_shift(sbits, bit) & 1
            e = 1.0 - 2.0 * m.astype(jnp.float32)
            return tuple(
                a + e * wp_v[pl.ds(((k * NCHUNK + c) * 32 + bit) * LANES, LANES)]
                for k, a in enumerate(accs))

        acc = list(lax.fori_loop(0, 32, clsbody, tuple(acc)))

    io = lax.broadcasted_iota(jnp.int32, (LANES,), 0)
    ov = jnp.zeros((LANES,), jnp.float32)
    for k in range(NCLS):
        total = acc[k][0]
        for l in range(1, LANES):
            total = total + acc[k][l]
        ov = jnp.where(io == k, total, ov)
    out_v[...] = ov
    pltpu.sync_copy(out_v, out.at[wid])


def _pack_signs(sgn):
    b = sgn.reshape(NLEV, NCHUNK, LANES, 32)
    w = jnp.sum(b << jnp.arange(32, dtype=jnp.uint32), axis=-1, dtype=jnp.uint32)
    return lax.bitcast_convert_type(w, jnp.int32)


def kernel(x, level_hv, W):
    batch = x.shape[0]
    flat = x.reshape(batch, -1)
    xin = jnp.pad(flat, ((0, 0), (0, XPAD - flat.shape[1])))

    sgn = (level_hv < 0).astype(jnp.uint32)
    ptab = jnp.stack([
        _pack_signs(jnp.roll(sgn, 2, axis=-1)),
        _pack_signs(jnp.roll(sgn, 1, axis=-1)),
        _pack_signs(sgn),
    ])
    wp = W.reshape(NCLS, NCHUNK, LANES, 32).transpose(0, 1, 3, 2)

    mesh = plsc.VectorSubcoreMesh(core_axis_name="c", subcore_axis_name="s")
    out = pl.kernel(
        _sc_body,
        mesh=mesh,
        out_type=jax.ShapeDtypeStruct((batch, LANES), jnp.float32),
        scratch_types=[
            pltpu.VMEM((XPAD,), jnp.float32),
            pltpu.VMEM((3 * NLEV * NCHUNK * LANES,), jnp.int32),
            pltpu.VMEM((NCLS * NCHUNK * 32 * LANES,), jnp.float32),
            pltpu.VMEM((LANES,), jnp.float32),
            pltpu.SMEM((XPAD,), jnp.int32),
        ],
    )(xin, ptab.reshape(-1), wp.reshape(-1))
    return out[:, :NCLS]


# speedup vs baseline: 19.1506x; 1.1418x over previous
import functools

import jax
import jax.numpy as jnp
from jax import lax
from jax.experimental import pallas as pl
from jax.experimental.pallas import tpu as pltpu
from jax.experimental.pallas import tpu_sc as plsc

DIM = 4096
NLEV = 21
NCHUNK = 8
LANES = 16
NJ = 598
NJ_GROUPS = 74
NJ_REM = 6
XPAD = 608
NCLS = 5
THRESH = 299


def _csa(a, b, cin):
    u = a ^ b
    return u ^ cin, (a & b) | (u & cin)


NWORD = NCHUNK * LANES


def _sc_body(xin, p0pad, wp, out, xin_v, p0pad_v, ptab_v, wp_v, out_v, offs_s):
    wid = lax.axis_index("s") * 2 + lax.axis_index("c")
    pltpu.sync_copy(xin.at[wid], xin_v)
    pltpu.sync_copy(p0pad, p0pad_v)
    pltpu.sync_copy(wp, wp_v)

    def mkrow(i, carry):
        for c in range(NCHUNK):
            a = p0pad_v[pl.ds(i * (NWORD + 1) + 1 + c * LANES, LANES)]
            ap = p0pad_v[pl.ds(i * (NWORD + 1) + c * LANES, LANES)]
            base = i * NWORD + c * LANES
            ptab_v[pl.ds(base, LANES)] = (
                (a << 2) | lax.shift_right_logical(ap, 30))
            ptab_v[pl.ds(NLEV * NWORD + base, LANES)] = (
                (a << 1) | lax.shift_right_logical(ap, 31))
            ptab_v[pl.ds(2 * NLEV * NWORD + base, LANES)] = a
        return carry

    lax.fori_loop(0, NLEV, mkrow, 0)

    for t in range(XPAD // LANES):
        v = xin_v[pl.ds(t * LANES, LANES)]
        u = (v / 20.0) * 20.0
        h = u + 0.5
        r = h.astype(jnp.int32)
        is_half = r.astype(jnp.float32) == h
        r = r - jnp.where(is_half, r & 1, 0)
        r = jnp.minimum(jnp.maximum(r, 0), NLEV - 1)
        roff = r * (NCHUNK * LANES)
        for l in range(LANES):
            offs_s[t * LANES + l] = roff[l]

    def product(ovals, k, c):
        w2 = ptab_v[pl.ds(ovals[k] + c * LANES, LANES)]
        w1 = ptab_v[pl.ds(ovals[k + 1] + (NLEV * NCHUNK + c) * LANES, LANES)]
        w0 = ptab_v[pl.ds(ovals[k + 2] + (2 * NLEV * NCHUNK + c) * LANES, LANES)]
        return (w2 ^ w1) ^ w0

    acc = [jnp.zeros((LANES,), jnp.float32) for _ in range(NCLS)]
    zero = jnp.zeros((LANES,), jnp.int32)

    for c in range(NCHUNK):
        def group(g, st, c=c):
            ones, twos, fours, p3, p4, p5, p6, p7, p8, p9 = st
            j0 = g * 8
            ovals = [offs_s[j0 + i] for i in range(10)]
            x = [product(ovals, k, c) for k in range(8)]
            s0, c0 = _csa(x[0], x[1], x[2])
            s1, c1 = _csa(x[3], x[4], x[5])
            s2, c2 = _csa(x[6], x[7], s0)
            ones, c3 = _csa(s1, s2, ones)
            t0, d0 = _csa(c0, c1, c2)
            twos, d1 = _csa(c3, t0, twos)
            fours, e0 = _csa(d0, d1, fours)
            carry = e0
            ps = [p3, p4, p5, p6, p7, p8, p9]
            for i in range(7):
                nxt = ps[i] ^ carry
                carry = ps[i] & carry
                ps[i] = nxt
            return (ones, twos, fours, *ps)

        planes = list(lax.fori_loop(0, NJ_GROUPS, group, (zero,) * 10))

        orem = [offs_s[NJ_GROUPS * 8 + i] for i in range(NJ_REM + 2)]
        for k in range(NJ_REM):
            carry = product(orem, k, c)
            for i in range(10):
                nxt = planes[i] ^ carry
                carry = planes[i] & carry
                planes[i] = nxt

        kadd = 1024 - THRESH
        carry = zero
        for p in range(10):
            if (kadd >> p) & 1:
                carry = planes[p] | carry
            else:
                carry = planes[p] & carry
        sbits = carry

        def clsbody(bit, accs, c=c, sbits=sbits):
            m = jnp.right_shift(sbits, bit) & 1
            e = 1.0 - 2.0 * m.astype(jnp.float32)
            return tuple(
                a + e * wp_v[pl.ds(((k * NCHUNK + c) * 32 + bit) * LANES, LANES)]
                for k, a in enumerate(accs))

        acc = list(lax.fori_loop(0, 32, clsbody, tuple(acc)))

    io = lax.broadcasted_iota(jnp.int32, (LANES,), 0)
    ov = jnp.zeros((LANES,), jnp.float32)
    for k in range(NCLS):
        total = acc[k][0]
        for l in range(1, LANES):
            total = total + acc[k][l]
        ov = jnp.where(io == k, total, ov)
    out_v[...] = ov
    pltpu.sync_copy(out_v, out.at[wid])


def _pack_signs(sgn):
    b = sgn.reshape(NLEV, NCHUNK, LANES, 32)
    w = jnp.sum(b << jnp.arange(32, dtype=jnp.uint32), axis=-1, dtype=jnp.uint32)
    return lax.bitcast_convert_type(w, jnp.int32)


def kernel(x, level_hv, W):
    batch = x.shape[0]
    flat = x.reshape(batch, -1)
    xin = jnp.pad(flat, ((0, 0), (0, XPAD - flat.shape[1])))

    sgn = (level_hv < 0).astype(jnp.uint32)
    p0 = _pack_signs(sgn).reshape(NLEV, NCHUNK * LANES)
    p0pad = jnp.concatenate([p0[:, -1:], p0], axis=1).reshape(-1)
    wp = W.reshape(NCLS, NCHUNK, LANES, 32).transpose(0, 1, 3, 2)

    mesh = plsc.VectorSubcoreMesh(core_axis_name="c", subcore_axis_name="s")
    out = pl.kernel(
        _sc_body,
        mesh=mesh,
        out_type=jax.ShapeDtypeStruct((batch, LANES), jnp.float32),
        scratch_types=[
            pltpu.VMEM((XPAD,), jnp.float32),
            pltpu.VMEM((NLEV * 129,), jnp.int32),
            pltpu.VMEM((3 * NLEV * NCHUNK * LANES,), jnp.int32),
            pltpu.VMEM((NCLS * NCHUNK * 32 * LANES,), jnp.float32),
            pltpu.VMEM((LANES,), jnp.float32),
            pltpu.SMEM((XPAD,), jnp.int32),
        ],
    )(xin, p0pad, wp.reshape(-1))
    return out[:, :NCLS]
